# baseline (speedup 1.0000x reference)
import functools

import jax
import jax.numpy as jnp
from jax.experimental import pallas as pl
from jax.experimental.pallas import tpu as pltpu

_FUDGE = 1e-07


def _stage_b_kernel(x_ref, w1_ref, adj_ref, w2_ref, o_ref, xw1_s):
    @pl.when(pl.program_id(0) == 0)
    def _():
        xw1_s[...] = jnp.dot(
            x_ref[...], w1_ref[...], preferred_element_type=jnp.float32
        )

    h = jnp.maximum(
        jnp.dot(adj_ref[...], xw1_s[...], preferred_element_type=jnp.float32), 0.0
    )
    o_ref[...] = jnp.dot(h, w2_ref[...], preferred_element_type=jnp.float32)


def _stage_c_kernel(adj_ref, hw2_ref, o_ref, *, l):
    acc = jnp.dot(adj_ref[...], hw2_ref[...], preferred_element_type=jnp.float32)
    o_ref[...] = acc[:, :l] + jnp.exp(acc[:, l:])


def _decoder_kernel(zr_ref, zc_ref, o_ref):
    p = jax.lax.dot_general(
        zr_ref[...],
        zc_ref[...],
        (((1,), (1,)), ((), ())),
        preferred_element_type=jnp.float32,
    )
    o_ref[...] = (jax.nn.sigmoid(p) + _FUDGE) * (1.0 - 2.0 * _FUDGE)


def _pick_bm(n):
    b = min(n, 400)
    while b > 8:
        if n % b == 0 and b % 8 == 0:
            return b
        b -= 8
    return n


def kernel(x, adj_norm, W1, W2_mu, W2_sig):
    n, d = x.shape
    h_dim = W1.shape[1]
    l_dim = W2_mu.shape[1]
    f32 = jnp.float32

    w2cat = jnp.concatenate([W2_mu, W2_sig], axis=1)

    bm = _pick_bm(n)
    nm = n // bm

    hw2 = pl.pallas_call(
        _stage_b_kernel,
        grid=(nm,),
        in_specs=[
            pl.BlockSpec((n, d), lambda i: (0, 0)),
            pl.BlockSpec((d, h_dim), lambda i: (0, 0)),
            pl.BlockSpec((bm, n), lambda i: (i, 0)),
            pl.BlockSpec((h_dim, 2 * l_dim), lambda i: (0, 0)),
        ],
        out_specs=pl.BlockSpec((bm, 2 * l_dim), lambda i: (i, 0)),
        out_shape=jax.ShapeDtypeStruct((n, 2 * l_dim), f32),
        scratch_shapes=[pltpu.VMEM((n, h_dim), f32)],
        compiler_params=pltpu.CompilerParams(
            dimension_semantics=(pltpu.ARBITRARY,)
        ),
    )(x, W1, adj_norm, w2cat)

    z = pl.pallas_call(
        functools.partial(_stage_c_kernel, l=l_dim),
        grid=(nm,),
        in_specs=[
            pl.BlockSpec((bm, n), lambda i: (i, 0)),
            pl.BlockSpec((n, 2 * l_dim), lambda i: (0, 0)),
        ],
        out_specs=pl.BlockSpec((bm, l_dim), lambda i: (i, 0)),
        out_shape=jax.ShapeDtypeStruct((n, l_dim), f32),
        compiler_params=pltpu.CompilerParams(
            dimension_semantics=(pltpu.PARALLEL,)
        ),
    )(adj_norm, hw2)

    adj_rec = pl.pallas_call(
        _decoder_kernel,
        grid=(nm,),
        in_specs=[
            pl.BlockSpec((bm, l_dim), lambda i: (i, 0)),
            pl.BlockSpec((n, l_dim), lambda i: (0, 0)),
        ],
        out_specs=pl.BlockSpec((bm, n), lambda i: (i, 0)),
        out_shape=jax.ShapeDtypeStruct((n, n), f32),
        compiler_params=pltpu.CompilerParams(
            dimension_semantics=(pltpu.PARALLEL,)
        ),
    )(z, z)

    return adj_rec


# speedup vs baseline: 1.0203x; 1.0203x over previous
import functools

import jax
import jax.numpy as jnp
from jax.experimental import pallas as pl
from jax.experimental.pallas import tpu as pltpu

_FUDGE = 1e-07


def _encoder_kernel(
    x_ref, w1_ref, adj_ref, w2_ref, z_ref, xw1_s, hw2_s, *, nm, bm, l
):
    i = pl.program_id(0)

    @pl.when(i == 0)
    def _():
        xw1_s[...] = jnp.dot(
            x_ref[...], w1_ref[...], preferred_element_type=jnp.float32
        )

    @pl.when(i < nm)
    def _():
        h = jnp.maximum(
            jnp.dot(adj_ref[...], xw1_s[...], preferred_element_type=jnp.float32),
            0.0,
        )
        hw2_s[pl.ds(i * bm, bm), :] = jnp.dot(
            h, w2_ref[...], preferred_element_type=jnp.float32
        )

    @pl.when(i >= nm)
    def _():
        acc = jnp.dot(
            adj_ref[...], hw2_s[...], preferred_element_type=jnp.float32
        )
        z_ref[...] = acc[:, :l] + jnp.exp(acc[:, l:])


def _decoder_kernel(zr_ref, zc_ref, o_ref):
    p = jax.lax.dot_general(
        zr_ref[...],
        zc_ref[...],
        (((1,), (1,)), ((), ())),
        preferred_element_type=jnp.float32,
    )
    o_ref[...] = (jax.nn.sigmoid(p) + _FUDGE) * (1.0 - 2.0 * _FUDGE)


def _pick_bm(n):
    b = min(n, 400)
    while b > 8:
        if n % b == 0 and b % 8 == 0:
            return b
        b -= 8
    return n


def kernel(x, adj_norm, W1, W2_mu, W2_sig):
    n, d = x.shape
    h_dim = W1.shape[1]
    l_dim = W2_mu.shape[1]
    f32 = jnp.float32

    w2cat = jnp.concatenate([W2_mu, W2_sig], axis=1)

    bm = _pick_bm(n)
    nm = n // bm

    z = pl.pallas_call(
        functools.partial(_encoder_kernel, nm=nm, bm=bm, l=l_dim),
        grid=(2 * nm,),
        in_specs=[
            pl.BlockSpec((n, d), lambda i: (0, 0)),
            pl.BlockSpec((d, h_dim), lambda i: (0, 0)),
            pl.BlockSpec((bm, n), lambda i: (jnp.where(i < nm, i, i - nm), 0)),
            pl.BlockSpec((h_dim, 2 * l_dim), lambda i: (0, 0)),
        ],
        out_specs=pl.BlockSpec((bm, l_dim), lambda i: (jnp.maximum(i - nm, 0), 0)),
        out_shape=jax.ShapeDtypeStruct((n, l_dim), f32),
        scratch_shapes=[
            pltpu.VMEM((n, h_dim), f32),
            pltpu.VMEM((n, 2 * l_dim), f32),
        ],
        compiler_params=pltpu.CompilerParams(
            dimension_semantics=(pltpu.ARBITRARY,)
        ),
    )(x, W1, adj_norm, w2cat)

    adj_rec = pl.pallas_call(
        _decoder_kernel,
        grid=(nm,),
        in_specs=[
            pl.BlockSpec((bm, l_dim), lambda i: (i, 0)),
            pl.BlockSpec((n, l_dim), lambda i: (0, 0)),
        ],
        out_specs=pl.BlockSpec((bm, n), lambda i: (i, 0)),
        out_shape=jax.ShapeDtypeStruct((n, n), f32),
        compiler_params=pltpu.CompilerParams(
            dimension_semantics=(pltpu.PARALLEL,)
        ),
    )(z, z)

    return adj_rec
